# baseline (speedup 1.0000x reference)
import jax
import jax.numpy as jnp
import numpy as np
from jax import lax
from jax.experimental import pallas as pl
from jax.experimental.pallas import tpu as pltpu
from jax.experimental.pallas import tpu_sc as plsc

B, H, L_Q, L_K, N_TOP = 4, 16, 4096, 4096, 64
R = B * H * N_TOP
GB = 32
ROW_W = L_K // 4
SEG_W = 2 * ROW_W
NC, NS = 2, 16
NW = NC * NS
RPW = R // NW
CHUNK = 32

_FE = np.concatenate([np.zeros(L_K, np.uint8), np.ones(L_K + GB, np.uint8)])
_TBL = np.concatenate(
    [np.frombuffer(_FE[g:g + 4 * SEG_W].tobytes(), dtype=np.int32)
     for g in range(GB)])


def _sc_body(idx_hbm, tbl_hbm, out_hbm, tbl_v, idx_v, sem):
    cid = lax.axis_index("c")
    sid = lax.axis_index("s")
    wid = sid * NC + cid
    base = wid * RPW

    pltpu.sync_copy(tbl_hbm, tbl_v)
    pltpu.sync_copy(idx_hbm.at[pl.ds(base, RPW)], idx_v)

    def chunk(c, carry):
        row0 = c * CHUNK
        copies = []
        for v in range(CHUNK // 16):
            vec = idx_v[pl.ds(row0 + v * 16, 16)]
            s_vec = (L_K - 1) - vec
            g_vec = lax.rem(s_vec, GB)
            o_vec = g_vec * SEG_W + lax.shift_right_logical(s_vec - g_vec, 2)
            for j in range(16):
                i = row0 + v * 16 + j
                o = pl.multiple_of(o_vec[j], 8)
                copies.append(pltpu.async_copy(
                    tbl_v.at[pl.ds(o, ROW_W)],
                    out_hbm.at[pl.ds((base + i) * ROW_W, ROW_W)], sem))
        for cp in copies:
            cp.wait()
        return carry

    lax.fori_loop(0, RPW // CHUNK, chunk, 0)


@jax.jit
def _sc_call(idx_flat, tbl):
    mesh = plsc.VectorSubcoreMesh(core_axis_name="c", subcore_axis_name="s")
    f = pl.kernel(
        _sc_body,
        out_type=jax.ShapeDtypeStruct((R * ROW_W,), jnp.int32),
        mesh=mesh,
        scratch_types=[
            pltpu.VMEM((GB * SEG_W,), jnp.int32),
            pltpu.VMEM((RPW,), jnp.int32),
            pltpu.SemaphoreType.DMA,
        ],
    )
    return f(idx_flat, tbl)


def kernel(index, scores):
    del scores
    idx_flat = index.reshape(R).astype(jnp.int32)
    out = _sc_call(idx_flat, jnp.asarray(_TBL))
    return out.view(jnp.int8).view(jnp.bool_).reshape(B, H, N_TOP, L_K)


# speedup vs baseline: 8.8095x; 8.8095x over previous
import jax
import jax.numpy as jnp
from jax.experimental import pallas as pl

B, H, L_Q, L_K, N_TOP = 4, 16, 4096, 4096, 64
BH = B * H


def _mask_kernel(idx_ref, out_ref):
    idx = idx_ref[...].astype(jnp.int16)
    idx = jnp.swapaxes(idx, 1, 2)
    col = jax.lax.broadcasted_iota(jnp.int16, out_ref.shape, 2)
    out_ref[...] = (col > idx).astype(jnp.int8)


def kernel(index, scores):
    del scores
    ROWS = 64
    idx3 = index.reshape(BH, 1, N_TOP).astype(jnp.int32)
    out = pl.pallas_call(
        _mask_kernel,
        grid=(BH // ROWS,),
        in_specs=[pl.BlockSpec((ROWS, 1, N_TOP), lambda i: (i, 0, 0))],
        out_specs=pl.BlockSpec((ROWS, N_TOP, L_K), lambda i: (i, 0, 0)),
        out_shape=jax.ShapeDtypeStruct((BH, N_TOP, L_K), jnp.int8),
    )(idx3)
    return out.view(jnp.bool_).reshape(B, H, N_TOP, L_K)


# speedup vs baseline: 8.9260x; 1.0132x over previous
import jax
import jax.numpy as jnp
from jax.experimental import pallas as pl

B, H, L_Q, L_K, N_TOP = 4, 16, 4096, 4096, 64
BH = B * H


def _mask_kernel(idx_ref, out_ref):
    idx = idx_ref[...].astype(jnp.int16)
    idx = jnp.swapaxes(idx, 1, 2)
    col = jax.lax.broadcasted_iota(jnp.int16, out_ref.shape, 2)
    out_ref[...] = (col > idx).astype(jnp.int8)


def kernel(index, scores):
    del scores
    ROWS = 8
    idx3 = index.reshape(BH, 1, N_TOP).astype(jnp.int32)
    out = pl.pallas_call(
        _mask_kernel,
        grid=(BH // ROWS,),
        in_specs=[pl.BlockSpec((ROWS, 1, N_TOP), lambda i: (i, 0, 0))],
        out_specs=pl.BlockSpec((ROWS, N_TOP, L_K), lambda i: (i, 0, 0)),
        out_shape=jax.ShapeDtypeStruct((BH, N_TOP, L_K), jnp.int8),
    )(idx3)
    return out.view(jnp.bool_).reshape(B, H, N_TOP, L_K)


# speedup vs baseline: 9.4028x; 1.0534x over previous
import jax
import jax.numpy as jnp
from jax.experimental import pallas as pl

B, H, L_Q, L_K, N_TOP = 4, 16, 4096, 4096, 64
BH = B * H


def _mask_kernel(idx_ref, out_ref):
    idx = idx_ref[...].astype(jnp.int16)
    idx = jnp.swapaxes(idx, 1, 2)
    col = jax.lax.broadcasted_iota(jnp.int16, out_ref.shape, 2)
    out_ref[...] = (col > idx).astype(jnp.int8)


def kernel(index, scores):
    del scores
    ROWS = 16
    idx3 = index.reshape(BH, 1, N_TOP).astype(jnp.int32)
    out = pl.pallas_call(
        _mask_kernel,
        grid=(BH // ROWS,),
        in_specs=[pl.BlockSpec((ROWS, 1, N_TOP), lambda i: (i, 0, 0))],
        out_specs=pl.BlockSpec((ROWS, N_TOP, L_K), lambda i: (i, 0, 0)),
        out_shape=jax.ShapeDtypeStruct((BH, N_TOP, L_K), jnp.int8),
    )(idx3)
    return out.view(jnp.bool_).reshape(B, H, N_TOP, L_K)
